# baseline (speedup 1.0000x reference)
import functools

import jax
import jax.numpy as jnp
from jax.experimental import pallas as pl
from jax.experimental.pallas import tpu as pltpu
from jax.experimental.pallas import tpu_sc as plsc


def _qbuild_body(x_ref, w2_ref, q_ref, wc_ref):
    xr = x_ref[...]
    m, H, W, C = xr.shape
    xj = jnp.concatenate([xr[:, :, 1:, :], xr[:, :, W - 1:W, :]], axis=2)
    xi = jnp.concatenate([xr[:, 1:, :, :], xr[:, H - 1:H, :, :]], axis=1)
    xij = jnp.concatenate([xi[:, :, 1:, :], xi[:, :, W - 1:W, :]], axis=2)
    q = jnp.concatenate([xr, xj, xi, xij], axis=3)
    q_ref[...] = q.reshape(m * H * W, 4 * C)

    N9, C_, F = w2_ref.shape
    wc_ref[...] = jnp.zeros(wc_ref.shape, wc_ref.dtype)
    w2 = w2_ref[...].astype(wc_ref.dtype)
    for c in range(4):
        wc_ref[:, c * C_:(c + 1) * C_, c * F:(c + 1) * F] = w2


def _sc_gather(q, idx, gwin):
    S = idx.shape[1]
    D = q.shape[1]
    mesh = plsc.VectorSubcoreMesh(core_axis_name="c", subcore_axis_name="s")

    @functools.partial(
        pl.kernel,
        out_type=jax.ShapeDtypeStruct((S, D), q.dtype),
        mesh=mesh,
    )
    def gather_kernel(q_hbm, i_hbm, o_hbm):
        def body(i_vmem, o_vmem):
            pltpu.sync_copy(q_hbm.at[i_vmem.at[0]], o_vmem)

        pltpu.emit_pipeline(
            body,
            grid=(S // gwin,),
            in_specs=[pl.BlockSpec((1, gwin), lambda i: (0, i))],
            out_specs=[pl.BlockSpec((gwin, D), lambda i: (i, 0))],
            core_axis_name=("c", "s"),
            dimension_semantics=(pltpu.PARALLEL,),
        )(i_hbm, o_hbm)

    return gather_kernel(q, idx)


def _combine_body(g_ref, fi_ref, fj_ref, w_ref, o_ref, acc_ref,
                  *, npg, nsteps, P, F):
    s = pl.program_id(0)
    lane = jax.lax.broadcasted_iota(jnp.int32, fi_ref.shape, 1)
    fi = fi_ref[...]
    fj = fj_ref[...]
    contrib = None
    for k in range(npg):
        gb = g_ref[pl.ds(k * P, P), :].astype(jnp.bfloat16)
        r = jnp.dot(gb, w_ref[k], preferred_element_type=jnp.float32)
        sel = (lane == s * npg + k).astype(jnp.float32)
        fin = jnp.sum(fi * sel, axis=1, keepdims=True)
        fjn = jnp.sum(fj * sel, axis=1, keepdims=True)
        c = ((1.0 - fin) * (1.0 - fjn) * r[:, 0:F]
             + (1.0 - fin) * fjn * r[:, F:2 * F]
             + fin * (1.0 - fjn) * r[:, 2 * F:3 * F]
             + fin * fjn * r[:, 3 * F:4 * F])
        contrib = c if contrib is None else contrib + c

    @pl.when(s == 0)
    def _():
        acc_ref[...] = contrib

    @pl.when(s > 0)
    def _():
        acc_ref[...] += contrib

    @pl.when(s == nsteps - 1)
    def _():
        o_ref[...] = acc_ref[...]


def kernel(input, offsets, W):
    x = input
    m, H, Wd, C = x.shape
    N9 = offsets.shape[3] // 2
    F = W.shape[2]
    P = m * H * Wd
    S = N9 * P
    gwin = 128
    s_pad = ((S + gwin - 1) // gwin) * gwin

    off = offsets.reshape(P, N9, 2)
    off_i = off[:, :, 0]
    off_j = off[:, :, 1]
    pos = jnp.arange(P, dtype=jnp.int32)[:, None]
    ii = (pos % (H * Wd)) // Wd
    jj = pos % Wd
    mb = pos // (H * Wd)
    ci = jnp.clip(ii.astype(jnp.float32) + off_i, 0.0, float(H - 1))
    cj = jnp.clip(jj.astype(jnp.float32) + off_j, 0.0, float(Wd - 1))
    lt_i = jnp.floor(ci)
    lt_j = jnp.floor(cj)
    fi = ci - lt_i
    fj = cj - lt_j
    idxmat = (mb * (H * Wd) + lt_i.astype(jnp.int32) * Wd
              + lt_j.astype(jnp.int32))
    idxT = idxmat.T

    w2 = W.transpose(1, 0, 2)
    q, w_comb = pl.pallas_call(
        _qbuild_body,
        out_shape=(jax.ShapeDtypeStruct((P, 4 * C), jnp.float32),
                   jax.ShapeDtypeStruct((N9, 4 * C, 4 * F), jnp.bfloat16)),
    )(x, w2)

    idx = jnp.pad(idxT.reshape(1, S), ((0, 0), (0, s_pad - S)))
    g = _sc_gather(q, idx, gwin)

    NPG = 9
    nsteps = N9 // NPG
    fspec = pl.BlockSpec((P, N9), lambda s: (0, 0))
    out = pl.pallas_call(
        functools.partial(_combine_body, npg=NPG, nsteps=nsteps, P=P, F=F),
        grid=(nsteps,),
        in_specs=[pl.BlockSpec((NPG * P, 4 * C), lambda s: (s, 0)),
                  fspec, fspec,
                  pl.BlockSpec((NPG, 4 * C, 4 * F), lambda s: (s, 0, 0))],
        out_specs=pl.BlockSpec((P, F), lambda s: (0, 0)),
        out_shape=jax.ShapeDtypeStruct((P, F), jnp.float32),
        scratch_shapes=[pltpu.VMEM((P, F), jnp.float32)],
    )(g, fi, fj, w_comb)

    return out.reshape(m, H, Wd, F)


# speedup vs baseline: 1.0500x; 1.0500x over previous
import functools

import jax
import jax.numpy as jnp
from jax.experimental import pallas as pl
from jax.experimental.pallas import tpu as pltpu
from jax.experimental.pallas import tpu_sc as plsc


def _qbuild_body(x_ref, w2_ref, q_ref, wc_ref):
    xr = x_ref[...]
    m, H, W, C = xr.shape
    xj = jnp.concatenate([xr[:, :, 1:, :], xr[:, :, W - 1:W, :]], axis=2)
    xi = jnp.concatenate([xr[:, 1:, :, :], xr[:, H - 1:H, :, :]], axis=1)
    xij = jnp.concatenate([xi[:, :, 1:, :], xi[:, :, W - 1:W, :]], axis=2)
    q = jnp.concatenate([xr, xj, xi, xij], axis=3)
    q_ref[...] = q.reshape(m * H * W, 4 * C)

    N9, C_, F = w2_ref.shape
    wc_ref[...] = jnp.zeros(wc_ref.shape, wc_ref.dtype)
    w2 = w2_ref[...].astype(wc_ref.dtype)
    for c in range(4):
        wc_ref[:, c * C_:(c + 1) * C_, c * F:(c + 1) * F] = w2


def _sc_gather(q, idx, gwin):
    S = idx.shape[1]
    D = q.shape[1]
    mesh = plsc.VectorSubcoreMesh(core_axis_name="c", subcore_axis_name="s")

    @functools.partial(
        pl.kernel,
        out_type=jax.ShapeDtypeStruct((S, D), q.dtype),
        mesh=mesh,
    )
    def gather_kernel(q_hbm, i_hbm, o_hbm):
        def body(i_vmem, o_vmem):
            pltpu.sync_copy(q_hbm.at[i_vmem.at[0]], o_vmem)

        pltpu.emit_pipeline(
            body,
            grid=(S // gwin,),
            in_specs=[pl.BlockSpec((1, gwin), lambda i: (0, i))],
            out_specs=[pl.BlockSpec((gwin, D), lambda i: (i, 0))],
            core_axis_name=("c", "s"),
            dimension_semantics=(pltpu.PARALLEL,),
        )(i_hbm, o_hbm)

    return gather_kernel(q, idx)


def _combine_body(g_ref, fi_ref, fj_ref, w_ref, o_ref, acc_ref,
                  *, npg, nsteps, P, F):
    s = pl.program_id(0)
    lane = jax.lax.broadcasted_iota(jnp.int32, fi_ref.shape, 1)
    fi = fi_ref[...]
    fj = fj_ref[...]
    contrib = None
    for k in range(npg):
        gb = g_ref[pl.ds(k * P, P), :].astype(jnp.bfloat16)
        r = jnp.dot(gb, w_ref[k], preferred_element_type=jnp.float32)
        sel = (lane == s * npg + k).astype(jnp.float32)
        fin = jnp.sum(fi * sel, axis=1, keepdims=True)
        fjn = jnp.sum(fj * sel, axis=1, keepdims=True)
        c = ((1.0 - fin) * (1.0 - fjn) * r[:, 0:F]
             + (1.0 - fin) * fjn * r[:, F:2 * F]
             + fin * (1.0 - fjn) * r[:, 2 * F:3 * F]
             + fin * fjn * r[:, 3 * F:4 * F])
        contrib = c if contrib is None else contrib + c

    @pl.when(s == 0)
    def _():
        acc_ref[...] = contrib

    @pl.when(s > 0)
    def _():
        acc_ref[...] += contrib

    @pl.when(s == nsteps - 1)
    def _():
        o_ref[...] = acc_ref[...]


def kernel(input, offsets, W):
    x = input
    m, H, Wd, C = x.shape
    N9 = offsets.shape[3] // 2
    F = W.shape[2]
    P = m * H * Wd
    S = N9 * P
    gwin = 128
    s_pad = ((S + gwin - 1) // gwin) * gwin

    off = offsets.reshape(P, N9, 2)
    off_i = off[:, :, 0]
    off_j = off[:, :, 1]
    pos = jnp.arange(P, dtype=jnp.int32)[:, None]
    ii = (pos % (H * Wd)) // Wd
    jj = pos % Wd
    mb = pos // (H * Wd)
    ci = jnp.clip(ii.astype(jnp.float32) + off_i, 0.0, float(H - 1))
    cj = jnp.clip(jj.astype(jnp.float32) + off_j, 0.0, float(Wd - 1))
    lt_i = jnp.floor(ci)
    lt_j = jnp.floor(cj)
    fi = ci - lt_i
    fj = cj - lt_j
    idxmat = (mb * (H * Wd) + lt_i.astype(jnp.int32) * Wd
              + lt_j.astype(jnp.int32))
    idxT = idxmat.T

    w2 = W.transpose(1, 0, 2)
    q, w_comb = pl.pallas_call(
        _qbuild_body,
        out_shape=(jax.ShapeDtypeStruct((P, 4 * C), jnp.float32),
                   jax.ShapeDtypeStruct((N9, 4 * C, 4 * F), jnp.bfloat16)),
    )(x, w2)

    idx = jnp.pad(idxT.reshape(1, S), ((0, 0), (0, s_pad - S)))
    g = _sc_gather(q, idx, gwin)

    NPG = 3
    nsteps = N9 // NPG
    fspec = pl.BlockSpec((P, N9), lambda s: (0, 0))
    out = pl.pallas_call(
        functools.partial(_combine_body, npg=NPG, nsteps=nsteps, P=P, F=F),
        grid=(nsteps,),
        in_specs=[pl.BlockSpec((NPG * P, 4 * C), lambda s: (s, 0)),
                  fspec, fspec,
                  pl.BlockSpec((NPG, 4 * C, 4 * F), lambda s: (s, 0, 0))],
        out_specs=pl.BlockSpec((P, F), lambda s: (0, 0)),
        out_shape=jax.ShapeDtypeStruct((P, F), jnp.float32),
        scratch_shapes=[pltpu.VMEM((P, F), jnp.float32)],
    )(g, fi, fj, w_comb)

    return out.reshape(m, H, Wd, F)
